# baseline (speedup 1.0000x reference)
import jax
import numpy as np
import jax.numpy as jnp
from jax.experimental import pallas as pl
from jax.experimental.pallas import tpu as pltpu

SIZE_HALF = 1024
N_SEG = 8
HIDDEN = 64
EPS = 1e-06
NPLANES = 2 * N_SEG + 1


def _spline_kernel(x_ref, w1_ref, b1_ref, w2_ref, b2_ref, phi_ref, ld_ref,
                   w2pm_ref):
    @pl.when(pl.program_id(0) == 0)
    def _():
        for j in range(NPLANES):
            w2pm_ref[:, j * SIZE_HALF:(j + 1) * SIZE_HALF] = (
                w2_ref[...].reshape(SIZE_HALF, NPLANES, HIDDEN)[:, j, :].T)

    x_a = x_ref[:, :SIZE_HALF]
    x_b = x_ref[:, SIZE_HALF:]

    hid = jnp.tanh(
        jnp.dot(x_a - 0.5, w1_ref[...], preferred_element_type=jnp.float32)
        + b1_ref[...]
    )
    out = jnp.tanh(
        jnp.dot(hid, w2pm_ref[...], preferred_element_type=jnp.float32)
        + b2_ref[...]
    )

    h_planes = [out[:, j * SIZE_HALF:(j + 1) * SIZE_HALF]
                for j in range(N_SEG + 1)]
    w_planes = [out[:, (N_SEG + 1 + j) * SIZE_HALF:(N_SEG + 2 + j) * SIZE_HALF]
                for j in range(N_SEG)]

    ew = [jnp.exp(p) for p in w_planes]
    S = ew[0]
    for t in ew[1:]:
        S = S + t

    eh = [jnp.exp(p) for p in h_planes]

    xbs = x_b * S
    xt = -EPS * S
    st = jnp.zeros_like(x_b)
    w_sel = ew[0]
    eh_sel = eh[0]
    ehp1_sel = eh[1]
    x_sel = xt
    s_sel = st
    for k in range(1, N_SEG):
        xt = xt + ew[k - 1]
        st = st + ew[k - 1] * (eh[k - 1] + eh[k])
        mask = xbs > xt
        w_sel = jnp.where(mask, ew[k], w_sel)
        eh_sel = jnp.where(mask, eh[k], eh_sel)
        ehp1_sel = jnp.where(mask, eh[k + 1], ehp1_sel)
        x_sel = jnp.where(mask, xt, x_sel)
        s_sel = jnp.where(mask, st, s_sel)
    dt2 = st + ew[N_SEG - 1] * (eh[N_SEG - 1] + eh[N_SEG])

    inv_dt2 = 1.0 / dt2
    alpha = (xbs - x_sel) / w_sel
    adh = alpha * (ehp1_sel - eh_sel)
    teh = eh_sel + eh_sel
    phi_b = (s_sel + alpha * w_sel * (teh + adh)) * inv_dt2
    grad = (teh + (adh + adh)) * (S * inv_dt2)

    phi_ref[:, :SIZE_HALF] = x_a
    phi_ref[:, SIZE_HALF:] = phi_b
    ld_ref[...] = -jnp.sum(jnp.log(grad), axis=1, keepdims=True)


def kernel(x_input, log_density, W1, b1, W2, b2):
    B = x_input.shape[0]
    W2t = W2.T
    b2r = b2.reshape(SIZE_HALF, NPLANES).T.reshape(1, NPLANES * SIZE_HALF)
    b1r = b1.reshape(1, HIDDEN)

    bb = 256
    grid = (B // bb,)
    phi, ld = pl.pallas_call(
        _spline_kernel,
        grid=grid,
        in_specs=[
            pl.BlockSpec((bb, 2 * SIZE_HALF), lambda i: (i, 0)),
            pl.BlockSpec((SIZE_HALF, HIDDEN), lambda i: (0, 0)),
            pl.BlockSpec((1, HIDDEN), lambda i: (0, 0)),
            pl.BlockSpec((NPLANES * SIZE_HALF, HIDDEN), lambda i: (0, 0)),
            pl.BlockSpec((1, NPLANES * SIZE_HALF), lambda i: (0, 0)),
        ],
        out_specs=[
            pl.BlockSpec((bb, 2 * SIZE_HALF), lambda i: (i, 0)),
            pl.BlockSpec((bb, 1), lambda i: (i, 0)),
        ],
        out_shape=[
            jax.ShapeDtypeStruct((B, 2 * SIZE_HALF), jnp.float32),
            jax.ShapeDtypeStruct((B, 1), jnp.float32),
        ],
        scratch_shapes=[
            pltpu.VMEM((HIDDEN, NPLANES * SIZE_HALF), jnp.float32),
        ],
        compiler_params=pltpu.CompilerParams(
            dimension_semantics=("arbitrary",),
        ),
    )(x_input, W1, b1r, W2t, b2r)
    return phi, log_density + ld


# speedup vs baseline: 1.0045x; 1.0045x over previous
import jax
import jax.numpy as jnp
from jax.experimental import pallas as pl
from jax.experimental.pallas import tpu as pltpu

SIZE_HALF = 1024
N_SEG = 8
HIDDEN = 64
EPS = 1e-06
NPLANES = 2 * N_SEG + 1


def _spline_kernel(x_ref, w1_ref, b1_ref, w2_ref, b2_ref, phi_ref, ld_ref,
                   w2pm_ref):
    @pl.when(pl.program_id(0) == 0)
    def _():
        for j in range(NPLANES):
            w2pm_ref[:, j * SIZE_HALF:(j + 1) * SIZE_HALF] = (
                w2_ref[...].reshape(SIZE_HALF, NPLANES, HIDDEN)[:, j, :].T)

    x_a = x_ref[:, :SIZE_HALF]
    x_b = x_ref[:, SIZE_HALF:]

    hid = jnp.tanh(
        jnp.dot(x_a - 0.5, w1_ref[...], preferred_element_type=jnp.float32)
        + b1_ref[...]
    )
    out = jnp.tanh(
        jnp.dot(hid, w2pm_ref[...], preferred_element_type=jnp.float32)
        + b2_ref[...]
    )

    h_planes = [out[:, j * SIZE_HALF:(j + 1) * SIZE_HALF]
                for j in range(N_SEG + 1)]
    w_planes = [out[:, (N_SEG + 1 + j) * SIZE_HALF:(N_SEG + 2 + j) * SIZE_HALF]
                for j in range(N_SEG)]

    ew = [jnp.exp(p) for p in w_planes]
    S = ew[0]
    for t in ew[1:]:
        S = S + t

    eh = [jnp.exp(p) for p in h_planes]

    xbs = x_b * S
    xt = -EPS * S
    st = jnp.zeros_like(x_b)
    w_sel = ew[0]
    eh_sel = eh[0]
    ehp1_sel = eh[1]
    x_sel = xt
    s_sel = st
    for k in range(1, N_SEG):
        xt = xt + ew[k - 1]
        st = st + ew[k - 1] * (eh[k - 1] + eh[k])
        mask = xbs > xt
        w_sel = jnp.where(mask, ew[k], w_sel)
        eh_sel = jnp.where(mask, eh[k], eh_sel)
        ehp1_sel = jnp.where(mask, eh[k + 1], ehp1_sel)
        x_sel = jnp.where(mask, xt, x_sel)
        s_sel = jnp.where(mask, st, s_sel)
    dt2 = st + ew[N_SEG - 1] * (eh[N_SEG - 1] + eh[N_SEG])

    inv_dt2 = 1.0 / dt2
    alpha = (xbs - x_sel) / w_sel
    adh = alpha * (ehp1_sel - eh_sel)
    teh = eh_sel + eh_sel
    phi_b = (s_sel + alpha * w_sel * (teh + adh)) * inv_dt2
    grad = (teh + (adh + adh)) * (S * inv_dt2)

    phi_ref[:, :SIZE_HALF] = x_a
    phi_ref[:, SIZE_HALF:] = phi_b
    ld_ref[...] = -jnp.sum(jnp.log(grad), axis=1, keepdims=True)


def kernel(x_input, log_density, W1, b1, W2, b2):
    B = x_input.shape[0]
    W2t = W2.T
    b2r = b2.reshape(SIZE_HALF, NPLANES).T.reshape(1, NPLANES * SIZE_HALF)
    b1r = b1.reshape(1, HIDDEN)

    bb = 256
    grid = (B // bb,)
    phi, ld = pl.pallas_call(
        _spline_kernel,
        grid=grid,
        in_specs=[
            pl.BlockSpec((bb, 2 * SIZE_HALF), lambda i: (i, 0)),
            pl.BlockSpec((SIZE_HALF, HIDDEN), lambda i: (0, 0)),
            pl.BlockSpec((1, HIDDEN), lambda i: (0, 0)),
            pl.BlockSpec((NPLANES * SIZE_HALF, HIDDEN), lambda i: (0, 0)),
            pl.BlockSpec((1, NPLANES * SIZE_HALF), lambda i: (0, 0)),
        ],
        out_specs=[
            pl.BlockSpec((bb, 2 * SIZE_HALF), lambda i: (i, 0)),
            pl.BlockSpec((bb, 1), lambda i: (i, 0)),
        ],
        out_shape=[
            jax.ShapeDtypeStruct((B, 2 * SIZE_HALF), jnp.float32),
            jax.ShapeDtypeStruct((B, 1), jnp.float32),
        ],
        scratch_shapes=[
            pltpu.VMEM((HIDDEN, NPLANES * SIZE_HALF), jnp.float32),
        ],
        compiler_params=pltpu.CompilerParams(
            dimension_semantics=("arbitrary",),
        ),
    )(x_input, W1, b1r, W2t, b2r)
    return phi, log_density + ld
